# baseline (speedup 1.0000x reference)
import functools

import jax
import jax.numpy as jnp
from jax import lax
from jax.experimental import pallas as pl
from jax.experimental.pallas import tpu as pltpu
from jax.experimental.pallas import tpu_sc as plsc

N_NODES = 10000
N_EDGES = 320000
D = 128
D_EDGE = 16
N_LAYERS = 5
N_GRAPHS = 64

NC = 2
NS = 16
EPC = N_EDGES // NC
WIN = 240
CHUNK = 80
N_PAD = 10240
RPT = N_PAD // NS
ZROWS = 64
NBUF = 64
PAD_ROW = N_PAD - 8

_NWIN = -(-EPC // WIN)
_WQ, _WR = divmod(_NWIN, NS)


def _tile_bounds(t):
    w = (_WQ + 1) * min(t, _WR) + _WQ * max(0, t - _WR)
    return min(WIN * w, EPC)


_BOUNDS = [_tile_bounds(t) for t in range(NS + 1)]


def _sc_agg_body(h_hbm, e_hbm, ssrc_hbm, seid_hbm, sdst_hbm, flg_hbm,
                 out_hbm,
                 sidx, eidx, didx, flags, hrows, erows, stage,
                 emitb, idemit, zbuf, agg_sh, sem0, sem1):
    c = lax.axis_index("c")
    t = lax.axis_index("s")
    lanes = lax.iota(jnp.int32, 16)
    zero16 = jnp.zeros((16,), jnp.float32)

    def _zrow(i, carry):
        for cc in range(D // 16):
            zbuf[i, pl.ds(cc * 16, 16)] = zero16
        return carry
    lax.fori_loop(0, ZROWS, _zrow, 0)
    for j in range(RPT // ZROWS):
        pltpu.sync_copy(zbuf, agg_sh.at[pl.ds(t * RPT + j * ZROWS, ZROWS)])
    def _zemit(i, carry):
        for cc in range(D // 16):
            emitb[i, pl.ds(cc * 16, 16)] = zero16
        return carry
    lax.fori_loop(0, 16, _zemit, 0)
    plsc.subcore_barrier()

    wq1 = _WQ + 1
    w = wq1 * jnp.minimum(t, _WR) + _WQ * jnp.maximum(t - _WR, 0)
    wn = wq1 * jnp.minimum(t + 1, _WR) + _WQ * jnp.maximum(t + 1 - _WR, 0)
    lo = c * EPC + jnp.minimum(WIN * w, EPC)
    hi = c * EPC + jnp.minimum(WIN * wn, EPC)
    nchunk = (hi - lo) // CHUNK

    def _chunk(ci, carry):
        newn, cur = carry[0], carry[1]
        accs0 = carry[2:]
        base = pl.multiple_of(lo + ci * CHUNK, 8)
        c1 = pltpu.async_copy(ssrc_hbm.at[pl.ds(base, CHUNK)], sidx, sem0)
        c2 = pltpu.async_copy(seid_hbm.at[pl.ds(base, CHUNK)], eidx, sem0)
        c3 = pltpu.async_copy(sdst_hbm.at[pl.ds(base, CHUNK)], didx, sem0)
        c4 = pltpu.async_copy(flg_hbm.at[pl.ds(base, CHUNK)],
                              flags.at[pl.ds(0, CHUNK)], sem0)
        c1.wait(); c2.wait(); c3.wait(); c4.wait()
        g1 = pltpu.async_copy(h_hbm.at[sidx], hrows, sem1)
        g2 = pltpu.async_copy(e_hbm.at[eidx], erows, sem1)
        g1.wait(); g2.wait()

        def _row(i, rc):
            nn = rc[0]
            accs = rc[1:]
            f = flags[pl.ds(i, 16)][0]
            newaccs = []
            for cc in range(D // 16):
                sl = pl.ds(cc * 16, 16)
                v = jnp.maximum(hrows[i, sl] + erows[i, sl], 0.0)
                na = jnp.where(nn != 0, v, accs[cc] + v)
                stage[i, sl] = jnp.where(f != 0, na, 0.0)
                newaccs.append(na)
            return (f,) + tuple(newaccs)

        out = lax.fori_loop(0, CHUNK, _row, (newn,) + accs0,
                            unroll=4)
        pltpu.sync_copy(stage, agg_sh.at[didx], add=True)
        curn = didx[pl.ds(CHUNK - 16, 16)][15]
        return (out[0], curn) + tuple(out[1:])

    fin = lax.fori_loop(0, nchunk, _chunk,
                        (jnp.int32(1), jnp.int32(PAD_ROW))
                        + tuple(zero16 for _ in range(8)))
    cur = fin[1]
    faccs = fin[2:]

    idemit[pl.ds(0, 16)] = jnp.full((16,), cur, jnp.int32)
    for cc in range(D // 16):
        emitb[0, pl.ds(cc * 16, 16)] = faccs[cc]
    pltpu.sync_copy(emitb, agg_sh.at[idemit], add=True)

    plsc.subcore_barrier()
    pltpu.sync_copy(agg_sh.at[pl.ds(t * RPT, RPT)],
                    out_hbm.at[c, pl.ds(t * RPT, RPT)])


@functools.cache
def _sc_agg_kernel():
    return pl.kernel(
        _sc_agg_body,
        out_type=jax.ShapeDtypeStruct((NC, N_PAD, D), jnp.float32),
        mesh=plsc.VectorSubcoreMesh(core_axis_name="c", subcore_axis_name="s",
                                    num_cores=NC, num_subcores=NS),
        scratch_types=[
            pltpu.VMEM((CHUNK,), jnp.int32),
            pltpu.VMEM((CHUNK,), jnp.int32),
            pltpu.VMEM((CHUNK,), jnp.int32),
            pltpu.VMEM((CHUNK + 16,), jnp.int32),
            pltpu.VMEM((CHUNK, D), jnp.float32),
            pltpu.VMEM((CHUNK, D), jnp.float32),
            pltpu.VMEM((CHUNK, D), jnp.float32),
            pltpu.VMEM((16, D), jnp.float32),
            pltpu.VMEM((16,), jnp.int32),
            pltpu.VMEM((ZROWS, D), jnp.float32),
            pltpu.VMEM_SHARED((N_PAD, D), jnp.float32),
            pltpu.SemaphoreType.DMA,
            pltpu.SemaphoreType.DMA,
        ],
    )


def _sc_agg(h, e, ssrc, seid, sdst, sflg):
    return _sc_agg_kernel()(h, e, ssrc, seid, sdst, sflg)


def kernel(x, edge_index, edge_attr, batch, We, be, Ws, bs, eps, gammas,
           betas, Wr1, br1, g_r, b_rn, Wr2, br2):
    src = edge_index[0]
    dst = edge_index[1]
    seid = jnp.argsort(dst, stable=True).astype(jnp.int32)
    sdst0 = jnp.take(dst, seid)
    ssrc = jnp.take(src, seid)
    nxt = jnp.concatenate([sdst0[1:], jnp.full((1,), -1, jnp.int32)])
    sflg = (sdst0 != nxt).astype(jnp.int32)
    ends = [c * EPC + b for c in range(NC) for b in _BOUNDS[1:]]
    sflg = sflg.at[jnp.asarray(ends, jnp.int32) - 1].set(0)
    sdst = jnp.pad(sdst0, (0, 32))
    sflg = jnp.pad(sflg, (0, 32))

    e = jax.nn.relu(edge_attr @ We + be)
    h = x
    for l in range(N_LAYERS):
        agg2 = _sc_agg(h, e, ssrc, seid, sdst, sflg)
        agg = (agg2[0] + agg2[1])[:N_NODES]
        hc = ((1.0 + eps[l]) * h + agg) @ Ws[l] + bs[l]
        mu = jnp.mean(hc, axis=0)
        var = jnp.var(hc, axis=0)
        hc = gammas[l] * (hc - mu) / jnp.sqrt(var + 1e-5) + betas[l]
        hc = jax.nn.relu(hc)
        h = hc + h
    ones = jnp.ones((N_NODES,), h.dtype)
    counts = jax.ops.segment_sum(ones, batch, num_segments=N_GRAPHS)
    hg = jax.ops.segment_sum(h, batch, num_segments=N_GRAPHS) / \
        jnp.maximum(counts, 1.0)[:, None]
    z = hg @ Wr1 + br1
    mu = jnp.mean(z, axis=0)
    var = jnp.var(z, axis=0)
    z = g_r * (z - mu) / jnp.sqrt(var + 1e-5) + b_rn
    z = jax.nn.relu(z)
    return z @ Wr2 + br2


# speedup vs baseline: 1.8334x; 1.8334x over previous
import functools

import jax
import jax.numpy as jnp
from jax import lax
from jax.experimental import pallas as pl
from jax.experimental.pallas import tpu as pltpu
from jax.experimental.pallas import tpu_sc as plsc

N_NODES = 10000
N_EDGES = 320000
D = 128
D_EDGE = 16
N_LAYERS = 5
N_GRAPHS = 64

NC = 2
NS = 16
EPC = N_EDGES // NC
WIN = 240
CHUNK = 80
N_PAD = 10240
RPT = N_PAD // NS
ZROWS = 64
NBUF = 64
PAD_ROW = N_PAD - 8

_NWIN = -(-EPC // WIN)
_WQ, _WR = divmod(_NWIN, NS)


def _tile_bounds(t):
    w = (_WQ + 1) * min(t, _WR) + _WQ * max(0, t - _WR)
    return min(WIN * w, EPC)


_BOUNDS = [_tile_bounds(t) for t in range(NS + 1)]


def _sc_agg_body(h_hbm, e_hbm, ssrc_hbm, seid_hbm, sdst_hbm, flg_hbm,
                 out_hbm,
                 sidx, eidx, didx, flags, hrows, erows, stage,
                 emitb, idemit, zbuf, agg_sh, sem0, sem1):
    c = lax.axis_index("c")
    t = lax.axis_index("s")
    lanes = lax.iota(jnp.int32, 16)
    zero16 = jnp.zeros((16,), jnp.float32)

    def _zrow(i, carry):
        for cc in range(D // 16):
            zbuf[i, pl.ds(cc * 16, 16)] = zero16
        return carry
    lax.fori_loop(0, ZROWS, _zrow, 0)
    for j in range(RPT // ZROWS):
        pltpu.sync_copy(zbuf, agg_sh.at[pl.ds(t * RPT + j * ZROWS, ZROWS)])
    def _zemit(i, carry):
        for cc in range(D // 16):
            emitb[i, pl.ds(cc * 16, 16)] = zero16
        return carry
    lax.fori_loop(0, 16, _zemit, 0)
    plsc.subcore_barrier()

    wq1 = _WQ + 1
    w = wq1 * jnp.minimum(t, _WR) + _WQ * jnp.maximum(t - _WR, 0)
    wn = wq1 * jnp.minimum(t + 1, _WR) + _WQ * jnp.maximum(t + 1 - _WR, 0)
    lo = c * EPC + jnp.minimum(WIN * w, EPC)
    hi = c * EPC + jnp.minimum(WIN * wn, EPC)
    nchunk = (hi - lo) // CHUNK

    def _chunk(ci, carry):
        newn, cur = carry[0], carry[1]
        accs0 = carry[2:]
        base = pl.multiple_of(lo + ci * CHUNK, 8)
        c1 = pltpu.async_copy(ssrc_hbm.at[pl.ds(base, CHUNK)], sidx, sem0)
        c2 = pltpu.async_copy(seid_hbm.at[pl.ds(base, CHUNK)], eidx, sem0)
        c3 = pltpu.async_copy(sdst_hbm.at[pl.ds(base, CHUNK)], didx, sem0)
        c4 = pltpu.async_copy(flg_hbm.at[pl.ds(base, CHUNK)],
                              flags.at[pl.ds(0, CHUNK)], sem0)
        c1.wait(); c2.wait(); c3.wait(); c4.wait()
        g1 = pltpu.async_copy(h_hbm.at[sidx], hrows, sem1)
        g2 = pltpu.async_copy(e_hbm.at[eidx], erows, sem1)
        g1.wait(); g2.wait()

        def _row(i, rc):
            nn = rc[0]
            accs = rc[1:]
            f = flags[pl.ds(i, 16)][0]
            newaccs = []
            for cc in range(D // 16):
                sl = pl.ds(cc * 16, 16)
                v = jnp.maximum(hrows[i, sl] + erows[i, sl], 0.0)
                na = jnp.where(nn != 0, v, accs[cc] + v)
                stage[i, sl] = jnp.where(f != 0, na, 0.0)
                newaccs.append(na)
            return (f,) + tuple(newaccs)

        out = lax.fori_loop(0, CHUNK, _row, (newn,) + accs0)
        pltpu.sync_copy(stage, agg_sh.at[didx], add=True)
        curn = didx[pl.ds(CHUNK - 16, 16)][15]
        return (out[0], curn) + tuple(out[1:])

    fin = lax.fori_loop(0, nchunk, _chunk,
                        (jnp.int32(1), jnp.int32(PAD_ROW))
                        + tuple(zero16 for _ in range(8)))
    cur = fin[1]
    faccs = fin[2:]

    idemit[pl.ds(0, 16)] = jnp.full((16,), cur, jnp.int32)
    for cc in range(D // 16):
        emitb[0, pl.ds(cc * 16, 16)] = faccs[cc]
    pltpu.sync_copy(emitb, agg_sh.at[idemit], add=True)

    plsc.subcore_barrier()
    pltpu.sync_copy(agg_sh.at[pl.ds(t * RPT, RPT)],
                    out_hbm.at[c, pl.ds(t * RPT, RPT)])


@functools.cache
def _sc_agg_kernel():
    return pl.kernel(
        _sc_agg_body,
        out_type=jax.ShapeDtypeStruct((NC, N_PAD, D), jnp.float32),
        mesh=plsc.VectorSubcoreMesh(core_axis_name="c", subcore_axis_name="s",
                                    num_cores=NC, num_subcores=NS),
        scratch_types=[
            pltpu.VMEM((CHUNK,), jnp.int32),
            pltpu.VMEM((CHUNK,), jnp.int32),
            pltpu.VMEM((CHUNK,), jnp.int32),
            pltpu.VMEM((CHUNK + 16,), jnp.int32),
            pltpu.VMEM((CHUNK, D), jnp.float32),
            pltpu.VMEM((CHUNK, D), jnp.float32),
            pltpu.VMEM((CHUNK, D), jnp.float32),
            pltpu.VMEM((16, D), jnp.float32),
            pltpu.VMEM((16,), jnp.int32),
            pltpu.VMEM((ZROWS, D), jnp.float32),
            pltpu.VMEM_SHARED((N_PAD, D), jnp.float32),
            pltpu.SemaphoreType.DMA,
            pltpu.SemaphoreType.DMA,
        ],
    )


def _sc_agg(h, e, ssrc, seid, sdst, sflg):
    return _sc_agg_kernel()(h, e, ssrc, seid, sdst, sflg)


def kernel(x, edge_index, edge_attr, batch, We, be, Ws, bs, eps, gammas,
           betas, Wr1, br1, g_r, b_rn, Wr2, br2):
    src = edge_index[0]
    dst = edge_index[1]
    seid = jnp.argsort(dst, stable=True).astype(jnp.int32)
    sdst0 = jnp.take(dst, seid)
    ssrc = jnp.take(src, seid)
    nxt = jnp.concatenate([sdst0[1:], jnp.full((1,), -1, jnp.int32)])
    sflg = (sdst0 != nxt).astype(jnp.int32)
    ends = [c * EPC + b for c in range(NC) for b in _BOUNDS[1:]]
    sflg = sflg.at[jnp.asarray(ends, jnp.int32) - 1].set(0)
    sdst = jnp.pad(sdst0, (0, 32))
    sflg = jnp.pad(sflg, (0, 32))

    e = jax.nn.relu(edge_attr @ We + be)
    h = x
    for l in range(N_LAYERS):
        agg2 = _sc_agg(h, e, ssrc, seid, sdst, sflg)
        agg = (agg2[0] + agg2[1])[:N_NODES]
        hc = ((1.0 + eps[l]) * h + agg) @ Ws[l] + bs[l]
        mu = jnp.mean(hc, axis=0)
        var = jnp.var(hc, axis=0)
        hc = gammas[l] * (hc - mu) / jnp.sqrt(var + 1e-5) + betas[l]
        hc = jax.nn.relu(hc)
        h = hc + h
    ones = jnp.ones((N_NODES,), h.dtype)
    counts = jax.ops.segment_sum(ones, batch, num_segments=N_GRAPHS)
    hg = jax.ops.segment_sum(h, batch, num_segments=N_GRAPHS) / \
        jnp.maximum(counts, 1.0)[:, None]
    z = hg @ Wr1 + br1
    mu = jnp.mean(z, axis=0)
    var = jnp.var(z, axis=0)
    z = g_r * (z - mu) / jnp.sqrt(var + 1e-5) + b_rn
    z = jax.nn.relu(z)
    return z @ Wr2 + br2
